# baseline (speedup 1.0000x reference)
import math
from functools import partial

import jax
import jax.numpy as jnp
from jax import lax
from jax.experimental import pallas as pl
from jax.experimental.pallas import tpu as pltpu

_INV_SQRT2 = 1.0 / math.sqrt(2.0)
_NEG_INF = -1e30
_HEADS = 12
_PATCH = 16
_EPS = 1e-5


def _ru(x, m):
    return ((x + m - 1) // m) * m


def _vmem_limit(bytes_needed):
    return int(min(64 * 2**20, max(32 * 2**20, 2 * bytes_needed)))


def _ln_rows(xv, g, b):
    mean = jnp.mean(xv, axis=-1, keepdims=True)
    xc = xv - mean
    var = jnp.mean(xc * xc, axis=-1, keepdims=True)
    return xc * lax.rsqrt(var + _EPS) * g + b


def _embed_kernel(x_ref, w_ref, b_ref, pos_ref, cls_ref, o_ref,
                  *, n_tok, n_pad, gh, p):
    xv = x_ref[0]
    c_in = xv.shape[0]
    p6 = xv.reshape(c_in, gh, p, gh, p)
    patches = p6.transpose(1, 3, 0, 2, 4).reshape(gh * gh, c_in * p * p)
    t = jnp.dot(patches.astype(jnp.bfloat16), w_ref[...],
                preferred_element_type=jnp.float32)
    t = jnp.pad(t, ((1, n_pad - n_tok), (0, 0)))
    y = t + b_ref[...] + pos_ref[...]
    rows = lax.broadcasted_iota(jnp.int32, (y.shape[0], 1), 0)
    y = jnp.where(rows == 0, cls_ref[...], y)
    y = jnp.where(rows >= n_tok, 0.0, y)
    o_ref[0] = y


def _mha(qkv, mask, *, nb, n_pad, heads, hd, scale):
    dim = heads * hd
    rows_out = []
    for bi in range(nb):
        r0 = bi * n_pad
        heads_out = []
        for h in range(heads):
            q = qkv[r0:r0 + n_pad, h * hd:(h + 1) * hd].astype(jnp.bfloat16)
            k = qkv[r0:r0 + n_pad,
                    dim + h * hd:dim + (h + 1) * hd].astype(jnp.bfloat16)
            v = qkv[r0:r0 + n_pad,
                    2 * dim + h * hd:2 * dim + (h + 1) * hd].astype(jnp.bfloat16)
            s = lax.dot_general(q, k, (((1,), (1,)), ((), ())),
                                preferred_element_type=jnp.float32) * scale
            s = jnp.where(mask, _NEG_INF, s)
            s = s - jnp.max(s, axis=-1, keepdims=True)
            p = jnp.exp(s)
            p = p / jnp.sum(p, axis=-1, keepdims=True)
            heads_out.append(jnp.dot(p.astype(jnp.bfloat16), v,
                                     preferred_element_type=jnp.float32))
        rows_out.append(jnp.concatenate(heads_out, axis=1))
    return jnp.concatenate(rows_out, axis=0)


def _attn_kernel(x_ref, g_ref, b_ref, wq_ref, bq_ref, o_ref, ln_ref, qkv_ref,
                 *, nb, n_pad, n_tok, heads, hd, scale):
    k = pl.program_id(1)

    @pl.when(k == 0)
    def _():
        xv = x_ref[...].reshape(nb * n_pad, x_ref.shape[-1])
        ln_ref[...] = _ln_rows(xv, g_ref[...], b_ref[...]).astype(jnp.bfloat16)

    sl = jnp.dot(ln_ref[...], wq_ref[...],
                 preferred_element_type=jnp.float32) + bq_ref[0]
    qkv_ref[k] = sl.astype(jnp.bfloat16)

    @pl.when(k == 2)
    def _():
        qv = qkv_ref[...]
        mask = lax.broadcasted_iota(jnp.int32, (n_pad, n_pad), 1) >= n_tok
        rows_out = []
        for bi in range(nb):
            r0 = bi * n_pad
            heads_out = []
            for h in range(heads):
                q = qv[0, r0:r0 + n_pad, h * hd:(h + 1) * hd]
                kk = qv[1, r0:r0 + n_pad, h * hd:(h + 1) * hd]
                v = qv[2, r0:r0 + n_pad, h * hd:(h + 1) * hd]
                s = lax.dot_general(q, kk, (((1,), (1,)), ((), ())),
                                    preferred_element_type=jnp.float32) * scale
                s = jnp.where(mask, _NEG_INF, s)
                s = s - jnp.max(s, axis=-1, keepdims=True)
                p = jnp.exp(s)
                p = p / jnp.sum(p, axis=-1, keepdims=True)
                heads_out.append(jnp.dot(p.astype(jnp.bfloat16), v,
                                         preferred_element_type=jnp.float32))
            rows_out.append(jnp.concatenate(heads_out, axis=1))
        o = jnp.concatenate(rows_out, axis=0)
        o_ref[...] = o.astype(jnp.bfloat16).reshape(o_ref.shape)


def _block_kernel(o_ref, x_ref, pw_ref, pb_ref, g_ref, b_ref,
                  w1_ref, b1_ref, w2_ref, b2_ref, out_ref,
                  xmid_ref, ln_ref, acc_ref):
    k = pl.program_id(1)

    @pl.when(k == 0)
    def _():
        rows = o_ref.shape[0] * o_ref.shape[1]
        ov = o_ref[...].reshape(rows, o_ref.shape[-1])
        xv = x_ref[...].reshape(rows, x_ref.shape[-1])
        t = jnp.dot(ov, pw_ref[...],
                    preferred_element_type=jnp.float32) + pb_ref[...]
        xmid = xv + t
        xmid_ref[...] = xmid
        ln_ref[...] = _ln_rows(xmid, g_ref[...], b_ref[...]).astype(jnp.bfloat16)
        acc_ref[...] = jnp.zeros_like(acc_ref)

    hh = jnp.dot(ln_ref[...], w1_ref[...],
                 preferred_element_type=jnp.float32) + b1_ref[0]
    gl = 0.5 * hh * (1.0 + lax.erf(hh * _INV_SQRT2))
    acc_ref[...] += jnp.dot(gl.astype(jnp.bfloat16), w2_ref[...],
                            preferred_element_type=jnp.float32)

    @pl.when(k == pl.num_programs(1) - 1)
    def _():
        out_ref[...] = (xmid_ref[...] + acc_ref[...]
                        + b2_ref[...]).reshape(out_ref.shape)


def _final_kernel(x_ref, g_ref, b_ref, o_ref):
    o_ref[...] = _ln_rows(x_ref[...], g_ref[...], b_ref[...])


def _row2d(a):
    return a.reshape(1, a.shape[-1]).astype(jnp.float32)


def kernel(patch_embed_w, patch_embed_b, cls_token, pos_embed, norm_g, norm_b, block0_ln1_g, block0_ln1_b, block0_qkv_w, block0_qkv_b, block0_proj_w, block0_proj_b, block0_ln2_g, block0_ln2_b, block0_fc1_w, block0_fc1_b, block0_fc2_w, block0_fc2_b, block1_ln1_g, block1_ln1_b, block1_qkv_w, block1_qkv_b, block1_proj_w, block1_proj_b, block1_ln2_g, block1_ln2_b, block1_fc1_w, block1_fc1_b, block1_fc2_w, block1_fc2_b, block2_ln1_g, block2_ln1_b, block2_qkv_w, block2_qkv_b, block2_proj_w, block2_proj_b, block2_ln2_g, block2_ln2_b, block2_fc1_w, block2_fc1_b, block2_fc2_w, block2_fc2_b, block3_ln1_g, block3_ln1_b, block3_qkv_w, block3_qkv_b, block3_proj_w, block3_proj_b, block3_ln2_g, block3_ln2_b, block3_fc1_w, block3_fc1_b, block3_fc2_w, block3_fc2_b, block4_ln1_g, block4_ln1_b, block4_qkv_w, block4_qkv_b, block4_proj_w, block4_proj_b, block4_ln2_g, block4_ln2_b, block4_fc1_w, block4_fc1_b, block4_fc2_w, block4_fc2_b, block5_ln1_g, block5_ln1_b, block5_qkv_w, block5_qkv_b, block5_proj_w, block5_proj_b, block5_ln2_g, block5_ln2_b, block5_fc1_w, block5_fc1_b, block5_fc2_w, block5_fc2_b, x):
    blocks = [
        (block0_ln1_g, block0_ln1_b, block0_qkv_w, block0_qkv_b, block0_proj_w,
         block0_proj_b, block0_ln2_g, block0_ln2_b, block0_fc1_w, block0_fc1_b,
         block0_fc2_w, block0_fc2_b),
        (block1_ln1_g, block1_ln1_b, block1_qkv_w, block1_qkv_b, block1_proj_w,
         block1_proj_b, block1_ln2_g, block1_ln2_b, block1_fc1_w, block1_fc1_b,
         block1_fc2_w, block1_fc2_b),
        (block2_ln1_g, block2_ln1_b, block2_qkv_w, block2_qkv_b, block2_proj_w,
         block2_proj_b, block2_ln2_g, block2_ln2_b, block2_fc1_w, block2_fc1_b,
         block2_fc2_w, block2_fc2_b),
        (block3_ln1_g, block3_ln1_b, block3_qkv_w, block3_qkv_b, block3_proj_w,
         block3_proj_b, block3_ln2_g, block3_ln2_b, block3_fc1_w, block3_fc1_b,
         block3_fc2_w, block3_fc2_b),
        (block4_ln1_g, block4_ln1_b, block4_qkv_w, block4_qkv_b, block4_proj_w,
         block4_proj_b, block4_ln2_g, block4_ln2_b, block4_fc1_w, block4_fc1_b,
         block4_fc2_w, block4_fc2_b),
        (block5_ln1_g, block5_ln1_b, block5_qkv_w, block5_qkv_b, block5_proj_w,
         block5_proj_b, block5_ln2_g, block5_ln2_b, block5_fc1_w, block5_fc1_b,
         block5_fc2_w, block5_fc2_b),
    ]

    B, C, IMG, _ = x.shape
    p = _PATCH
    gh = IMG // p
    n_patch = gh * gh
    n_tok = n_patch + 1
    n_pad = _ru(n_tok, 8)
    D = patch_embed_w.shape[1]
    K = C * p * p
    H = _HEADS
    hd = D // H
    hidden = blocks[0][8].shape[1]
    scale = hd ** -0.5

    pos = pos_embed[0].astype(jnp.float32)
    pos_pad = jnp.pad(pos, ((0, n_pad - n_tok), (0, 0)))
    cls0 = (cls_token[0, 0] + pos[0]).reshape(1, D).astype(jnp.float32)

    xs = pl.pallas_call(
        partial(_embed_kernel, n_tok=n_tok, n_pad=n_pad, gh=gh, p=p),
        out_shape=jax.ShapeDtypeStruct((B, n_pad, D), jnp.float32),
        grid_spec=pltpu.PrefetchScalarGridSpec(
            num_scalar_prefetch=0,
            grid=(B,),
            in_specs=[
                pl.BlockSpec((1, C, IMG, IMG), lambda i: (i, 0, 0, 0)),
                pl.BlockSpec((K, D), lambda i: (0, 0)),
                pl.BlockSpec((1, D), lambda i: (0, 0)),
                pl.BlockSpec((n_pad, D), lambda i: (0, 0)),
                pl.BlockSpec((1, D), lambda i: (0, 0)),
            ],
            out_specs=pl.BlockSpec((1, n_pad, D), lambda i: (i, 0, 0)),
        ),
        compiler_params=pltpu.CompilerParams(
            dimension_semantics=("parallel",),
            vmem_limit_bytes=_vmem_limit(4 * (K * D + 3 * n_pad * D + C * IMG * IMG))),
    )(x, patch_embed_w.astype(jnp.bfloat16), _row2d(patch_embed_b), pos_pad, cls0)

    NB_A = min(4, B)
    NB_B = min(4, B)
    KS = 4
    ts = hidden // KS
    aspec = pl.BlockSpec((NB_A, n_pad, D), lambda i, k: (i, 0, 0))
    arow = pl.BlockSpec((1, D), lambda i, k: (0, 0))
    bspec = pl.BlockSpec((NB_B, n_pad, D), lambda i, k: (i, 0, 0))
    brow = pl.BlockSpec((1, D), lambda i, k: (0, 0))
    for (ln1_g, ln1_b, qkv_w, qkv_b, proj_w, proj_b,
         ln2_g, ln2_b, fc1_w, fc1_b, fc2_w, fc2_b) in blocks:
        o_t = pl.pallas_call(
            partial(_attn_kernel, nb=NB_A, n_pad=n_pad, n_tok=n_tok,
                    heads=H, hd=hd, scale=scale),
            out_shape=jax.ShapeDtypeStruct((B, n_pad, D), jnp.bfloat16),
            grid_spec=pltpu.PrefetchScalarGridSpec(
                num_scalar_prefetch=0,
                grid=(B // NB_A, 3),
                in_specs=[
                    aspec,
                    arow,
                    arow,
                    pl.BlockSpec((D, D), lambda i, k: (0, k)),
                    pl.BlockSpec((1, 1, D), lambda i, k: (k, 0, 0)),
                ],
                out_specs=aspec,
                scratch_shapes=[
                    pltpu.VMEM((NB_A * n_pad, D), jnp.bfloat16),
                    pltpu.VMEM((3, NB_A * n_pad, D), jnp.bfloat16),
                ],
            ),
            compiler_params=pltpu.CompilerParams(
                dimension_semantics=("parallel", "arbitrary"),
                vmem_limit_bytes=_vmem_limit(
                    2 * 2 * D * D + 10 * NB_A * n_pad * D
                    + 4 * NB_A * (2 * n_pad * D + 2 * n_pad * n_pad))),
        )(xs, _row2d(ln1_g), _row2d(ln1_b),
          qkv_w.astype(jnp.bfloat16), qkv_b.reshape(3, 1, D).astype(jnp.float32))
        xs = pl.pallas_call(
            _block_kernel,
            out_shape=jax.ShapeDtypeStruct((B, n_pad, D), jnp.float32),
            grid_spec=pltpu.PrefetchScalarGridSpec(
                num_scalar_prefetch=0,
                grid=(B // NB_B, KS),
                in_specs=[
                    bspec,
                    bspec,
                    pl.BlockSpec((D, D), lambda i, k: (0, 0)),
                    brow,
                    brow,
                    brow,
                    pl.BlockSpec((D, ts), lambda i, k: (0, k)),
                    pl.BlockSpec((1, 1, ts), lambda i, k: (k, 0, 0)),
                    pl.BlockSpec((ts, D), lambda i, k: (k, 0)),
                    brow,
                ],
                out_specs=pl.BlockSpec((NB_B, n_pad, D), lambda i, k: (i, 0, 0)),
                scratch_shapes=[
                    pltpu.VMEM((NB_B * n_pad, D), jnp.float32),
                    pltpu.VMEM((NB_B * n_pad, D), jnp.bfloat16),
                    pltpu.VMEM((NB_B * n_pad, D), jnp.float32),
                ],
            ),
            compiler_params=pltpu.CompilerParams(
                dimension_semantics=("parallel", "arbitrary"),
                vmem_limit_bytes=_vmem_limit(
                    2 * (D * D + 2 * D * ts) + 10 * NB_B * n_pad * D
                    + 4 * NB_B * (2 * n_pad * D + n_pad * ts))),
        )(o_t, xs, proj_w.astype(jnp.bfloat16), _row2d(proj_b),
          _row2d(ln2_g), _row2d(ln2_b), fc1_w.astype(jnp.bfloat16),
          fc1_b.reshape(KS, 1, ts).astype(jnp.float32),
          fc2_w.astype(jnp.bfloat16), _row2d(fc2_b))

    cls_rows = xs[:, 0, :]
    out = pl.pallas_call(
        _final_kernel,
        out_shape=jax.ShapeDtypeStruct((B, D), jnp.float32),
        grid_spec=pltpu.PrefetchScalarGridSpec(
            num_scalar_prefetch=0,
            grid=(1,),
            in_specs=[
                pl.BlockSpec((B, D), lambda i: (0, 0)),
                pl.BlockSpec((1, D), lambda i: (0, 0)),
                pl.BlockSpec((1, D), lambda i: (0, 0)),
            ],
            out_specs=pl.BlockSpec((B, D), lambda i: (0, 0)),
        ),
        compiler_params=pltpu.CompilerParams(
            dimension_semantics=("arbitrary",)),
    )(cls_rows, _row2d(norm_g), _row2d(norm_b))
    return out


# speedup vs baseline: 1.1062x; 1.1062x over previous
import math
from functools import partial

import jax
import jax.numpy as jnp
from jax import lax
from jax.experimental import pallas as pl
from jax.experimental.pallas import tpu as pltpu

_INV_SQRT2 = 1.0 / math.sqrt(2.0)
_NEG_INF = -1e30
_HEADS = 12
_PATCH = 16
_EPS = 1e-5


def _ru(x, m):
    return ((x + m - 1) // m) * m


def _vmem_limit(bytes_needed):
    return int(min(64 * 2**20, max(32 * 2**20, 2 * bytes_needed)))


def _ln_rows(xv, g, b):
    mean = jnp.mean(xv, axis=-1, keepdims=True)
    xc = xv - mean
    var = jnp.mean(xc * xc, axis=-1, keepdims=True)
    return xc * lax.rsqrt(var + _EPS) * g + b


def _embed_kernel(x_ref, w_ref, b_ref, pos_ref, cls_ref, o_ref,
                  *, n_tok, n_pad, gh, p):
    xv = x_ref[0]
    c_in = xv.shape[0]
    p6 = xv.reshape(c_in, gh, p, gh, p)
    patches = p6.transpose(1, 3, 0, 2, 4).reshape(gh * gh, c_in * p * p)
    t = jnp.dot(patches.astype(jnp.bfloat16), w_ref[...],
                preferred_element_type=jnp.float32)
    t = jnp.pad(t, ((1, n_pad - n_tok), (0, 0)))
    y = t + b_ref[...] + pos_ref[...]
    rows = lax.broadcasted_iota(jnp.int32, (y.shape[0], 1), 0)
    y = jnp.where(rows == 0, cls_ref[...], y)
    y = jnp.where(rows >= n_tok, 0.0, y)
    o_ref[0] = y


def _mha(qkv, mask, *, nb, n_pad, heads, hd, scale):
    dim = heads * hd
    rows_out = []
    for bi in range(nb):
        r0 = bi * n_pad
        heads_out = []
        for h in range(heads):
            q = qkv[r0:r0 + n_pad, h * hd:(h + 1) * hd].astype(jnp.bfloat16)
            k = qkv[r0:r0 + n_pad,
                    dim + h * hd:dim + (h + 1) * hd].astype(jnp.bfloat16)
            v = qkv[r0:r0 + n_pad,
                    2 * dim + h * hd:2 * dim + (h + 1) * hd].astype(jnp.bfloat16)
            s = lax.dot_general(q, k, (((1,), (1,)), ((), ())),
                                preferred_element_type=jnp.float32) * scale
            s = jnp.where(mask, _NEG_INF, s)
            s = s - jnp.max(s, axis=-1, keepdims=True)
            p = jnp.exp(s)
            p = p / jnp.sum(p, axis=-1, keepdims=True)
            heads_out.append(jnp.dot(p.astype(jnp.bfloat16), v,
                                     preferred_element_type=jnp.float32))
        rows_out.append(jnp.concatenate(heads_out, axis=1))
    return jnp.concatenate(rows_out, axis=0)


def _attn_kernel(x_ref, g_ref, b_ref, wq_ref, bq_ref, o_ref,
                 *, nb, n_pad, n_tok, heads, hd, scale):
    xv = x_ref[...].reshape(nb * n_pad, x_ref.shape[-1])
    ln = _ln_rows(xv, g_ref[...], b_ref[...]).astype(jnp.bfloat16)
    qkv = jnp.dot(ln, wq_ref[...], preferred_element_type=jnp.float32)
    qkv = qkv + bq_ref[...]
    mask = lax.broadcasted_iota(jnp.int32, (n_pad, n_pad), 1) >= n_tok
    o = _mha(qkv, mask, nb=nb, n_pad=n_pad, heads=heads, hd=hd, scale=scale)
    o_ref[...] = o.astype(jnp.bfloat16).reshape(o_ref.shape)


def _block_kernel(o_ref, x_ref, pw_ref, pb_ref, g_ref, b_ref,
                  w1_ref, b1_ref, w2_ref, b2_ref, out_ref,
                  xmid_ref, ln_ref, acc_ref):
    k = pl.program_id(1)

    @pl.when(k == 0)
    def _():
        rows = o_ref.shape[0] * o_ref.shape[1]
        ov = o_ref[...].reshape(rows, o_ref.shape[-1])
        xv = x_ref[...].reshape(rows, x_ref.shape[-1])
        t = jnp.dot(ov, pw_ref[...],
                    preferred_element_type=jnp.float32) + pb_ref[...]
        xmid = xv + t
        xmid_ref[...] = xmid
        ln_ref[...] = _ln_rows(xmid, g_ref[...], b_ref[...]).astype(jnp.bfloat16)
        acc_ref[...] = jnp.zeros_like(acc_ref)

    hh = jnp.dot(ln_ref[...], w1_ref[...],
                 preferred_element_type=jnp.float32) + b1_ref[0]
    gl = 0.5 * hh * (1.0 + lax.erf(hh * _INV_SQRT2))
    acc_ref[...] += jnp.dot(gl.astype(jnp.bfloat16), w2_ref[...],
                            preferred_element_type=jnp.float32)

    @pl.when(k == pl.num_programs(1) - 1)
    def _():
        out_ref[...] = (xmid_ref[...] + acc_ref[...]
                        + b2_ref[...]).reshape(out_ref.shape)


def _final_kernel(x_ref, g_ref, b_ref, o_ref):
    o_ref[...] = _ln_rows(x_ref[...], g_ref[...], b_ref[...])


def _row2d(a):
    return a.reshape(1, a.shape[-1]).astype(jnp.float32)


def kernel(patch_embed_w, patch_embed_b, cls_token, pos_embed, norm_g, norm_b, block0_ln1_g, block0_ln1_b, block0_qkv_w, block0_qkv_b, block0_proj_w, block0_proj_b, block0_ln2_g, block0_ln2_b, block0_fc1_w, block0_fc1_b, block0_fc2_w, block0_fc2_b, block1_ln1_g, block1_ln1_b, block1_qkv_w, block1_qkv_b, block1_proj_w, block1_proj_b, block1_ln2_g, block1_ln2_b, block1_fc1_w, block1_fc1_b, block1_fc2_w, block1_fc2_b, block2_ln1_g, block2_ln1_b, block2_qkv_w, block2_qkv_b, block2_proj_w, block2_proj_b, block2_ln2_g, block2_ln2_b, block2_fc1_w, block2_fc1_b, block2_fc2_w, block2_fc2_b, block3_ln1_g, block3_ln1_b, block3_qkv_w, block3_qkv_b, block3_proj_w, block3_proj_b, block3_ln2_g, block3_ln2_b, block3_fc1_w, block3_fc1_b, block3_fc2_w, block3_fc2_b, block4_ln1_g, block4_ln1_b, block4_qkv_w, block4_qkv_b, block4_proj_w, block4_proj_b, block4_ln2_g, block4_ln2_b, block4_fc1_w, block4_fc1_b, block4_fc2_w, block4_fc2_b, block5_ln1_g, block5_ln1_b, block5_qkv_w, block5_qkv_b, block5_proj_w, block5_proj_b, block5_ln2_g, block5_ln2_b, block5_fc1_w, block5_fc1_b, block5_fc2_w, block5_fc2_b, x):
    blocks = [
        (block0_ln1_g, block0_ln1_b, block0_qkv_w, block0_qkv_b, block0_proj_w,
         block0_proj_b, block0_ln2_g, block0_ln2_b, block0_fc1_w, block0_fc1_b,
         block0_fc2_w, block0_fc2_b),
        (block1_ln1_g, block1_ln1_b, block1_qkv_w, block1_qkv_b, block1_proj_w,
         block1_proj_b, block1_ln2_g, block1_ln2_b, block1_fc1_w, block1_fc1_b,
         block1_fc2_w, block1_fc2_b),
        (block2_ln1_g, block2_ln1_b, block2_qkv_w, block2_qkv_b, block2_proj_w,
         block2_proj_b, block2_ln2_g, block2_ln2_b, block2_fc1_w, block2_fc1_b,
         block2_fc2_w, block2_fc2_b),
        (block3_ln1_g, block3_ln1_b, block3_qkv_w, block3_qkv_b, block3_proj_w,
         block3_proj_b, block3_ln2_g, block3_ln2_b, block3_fc1_w, block3_fc1_b,
         block3_fc2_w, block3_fc2_b),
        (block4_ln1_g, block4_ln1_b, block4_qkv_w, block4_qkv_b, block4_proj_w,
         block4_proj_b, block4_ln2_g, block4_ln2_b, block4_fc1_w, block4_fc1_b,
         block4_fc2_w, block4_fc2_b),
        (block5_ln1_g, block5_ln1_b, block5_qkv_w, block5_qkv_b, block5_proj_w,
         block5_proj_b, block5_ln2_g, block5_ln2_b, block5_fc1_w, block5_fc1_b,
         block5_fc2_w, block5_fc2_b),
    ]

    B, C, IMG, _ = x.shape
    p = _PATCH
    gh = IMG // p
    n_patch = gh * gh
    n_tok = n_patch + 1
    n_pad = _ru(n_tok, 8)
    D = patch_embed_w.shape[1]
    K = C * p * p
    H = _HEADS
    hd = D // H
    hidden = blocks[0][8].shape[1]
    scale = hd ** -0.5

    pos = pos_embed[0].astype(jnp.float32)
    pos_pad = jnp.pad(pos, ((0, n_pad - n_tok), (0, 0)))
    cls0 = (cls_token[0, 0] + pos[0]).reshape(1, D).astype(jnp.float32)

    xs = pl.pallas_call(
        partial(_embed_kernel, n_tok=n_tok, n_pad=n_pad, gh=gh, p=p),
        out_shape=jax.ShapeDtypeStruct((B, n_pad, D), jnp.float32),
        grid_spec=pltpu.PrefetchScalarGridSpec(
            num_scalar_prefetch=0,
            grid=(B,),
            in_specs=[
                pl.BlockSpec((1, C, IMG, IMG), lambda i: (i, 0, 0, 0)),
                pl.BlockSpec((K, D), lambda i: (0, 0)),
                pl.BlockSpec((1, D), lambda i: (0, 0)),
                pl.BlockSpec((n_pad, D), lambda i: (0, 0)),
                pl.BlockSpec((1, D), lambda i: (0, 0)),
            ],
            out_specs=pl.BlockSpec((1, n_pad, D), lambda i: (i, 0, 0)),
        ),
        compiler_params=pltpu.CompilerParams(
            dimension_semantics=("parallel",),
            vmem_limit_bytes=_vmem_limit(4 * (K * D + 3 * n_pad * D + C * IMG * IMG))),
    )(x, patch_embed_w.astype(jnp.bfloat16), _row2d(patch_embed_b), pos_pad, cls0)

    NB_A = min(2, B)
    NB_B = min(4, B)
    KS = 8
    ts = hidden // KS
    aspec = pl.BlockSpec((NB_A, n_pad, D), lambda i: (i, 0, 0))
    arow = pl.BlockSpec((1, D), lambda i: (0, 0))
    bspec = pl.BlockSpec((NB_B, n_pad, D), lambda i, k: (i, 0, 0))
    brow = pl.BlockSpec((1, D), lambda i, k: (0, 0))
    for (ln1_g, ln1_b, qkv_w, qkv_b, proj_w, proj_b,
         ln2_g, ln2_b, fc1_w, fc1_b, fc2_w, fc2_b) in blocks:
        o_t = pl.pallas_call(
            partial(_attn_kernel, nb=NB_A, n_pad=n_pad, n_tok=n_tok,
                    heads=H, hd=hd, scale=scale),
            out_shape=jax.ShapeDtypeStruct((B, n_pad, D), jnp.bfloat16),
            grid_spec=pltpu.PrefetchScalarGridSpec(
                num_scalar_prefetch=0,
                grid=(B // NB_A,),
                in_specs=[
                    aspec,
                    arow,
                    arow,
                    pl.BlockSpec((D, 3 * D), lambda i: (0, 0)),
                    pl.BlockSpec((1, 3 * D), lambda i: (0, 0)),
                ],
                out_specs=aspec,
            ),
            compiler_params=pltpu.CompilerParams(
                dimension_semantics=("parallel",),
                vmem_limit_bytes=_vmem_limit(
                    2 * D * 3 * D + 4 * NB_A * (3 * n_pad * D + n_pad * 3 * D
                                                + 2 * n_pad * n_pad))),
        )(xs, _row2d(ln1_g), _row2d(ln1_b),
          qkv_w.astype(jnp.bfloat16), _row2d(qkv_b))
        xs = pl.pallas_call(
            _block_kernel,
            out_shape=jax.ShapeDtypeStruct((B, n_pad, D), jnp.float32),
            grid_spec=pltpu.PrefetchScalarGridSpec(
                num_scalar_prefetch=0,
                grid=(B // NB_B, KS),
                in_specs=[
                    bspec,
                    bspec,
                    pl.BlockSpec((D, D), lambda i, k: (0, 0)),
                    brow,
                    brow,
                    brow,
                    pl.BlockSpec((D, ts), lambda i, k: (0, k)),
                    pl.BlockSpec((1, 1, ts), lambda i, k: (k, 0, 0)),
                    pl.BlockSpec((ts, D), lambda i, k: (k, 0)),
                    brow,
                ],
                out_specs=pl.BlockSpec((NB_B, n_pad, D), lambda i, k: (i, 0, 0)),
                scratch_shapes=[
                    pltpu.VMEM((NB_B * n_pad, D), jnp.float32),
                    pltpu.VMEM((NB_B * n_pad, D), jnp.bfloat16),
                    pltpu.VMEM((NB_B * n_pad, D), jnp.float32),
                ],
            ),
            compiler_params=pltpu.CompilerParams(
                dimension_semantics=("parallel", "arbitrary"),
                vmem_limit_bytes=_vmem_limit(
                    2 * (D * D + 2 * D * ts) + 10 * NB_B * n_pad * D
                    + 4 * NB_B * (2 * n_pad * D + n_pad * ts))),
        )(o_t, xs, proj_w.astype(jnp.bfloat16), _row2d(proj_b),
          _row2d(ln2_g), _row2d(ln2_b), fc1_w.astype(jnp.bfloat16),
          fc1_b.reshape(KS, 1, ts).astype(jnp.float32),
          fc2_w.astype(jnp.bfloat16), _row2d(fc2_b))

    cls_rows = xs[:, 0, :]
    out = pl.pallas_call(
        _final_kernel,
        out_shape=jax.ShapeDtypeStruct((B, D), jnp.float32),
        grid_spec=pltpu.PrefetchScalarGridSpec(
            num_scalar_prefetch=0,
            grid=(1,),
            in_specs=[
                pl.BlockSpec((B, D), lambda i: (0, 0)),
                pl.BlockSpec((1, D), lambda i: (0, 0)),
                pl.BlockSpec((1, D), lambda i: (0, 0)),
            ],
            out_specs=pl.BlockSpec((B, D), lambda i: (0, 0)),
        ),
        compiler_params=pltpu.CompilerParams(
            dimension_semantics=("arbitrary",)),
    )(cls_rows, _row2d(norm_g), _row2d(norm_b))
    return out


# speedup vs baseline: 1.3417x; 1.2129x over previous
import math
from functools import partial

import jax
import jax.numpy as jnp
from jax import lax
from jax.experimental import pallas as pl
from jax.experimental.pallas import tpu as pltpu

_INV_SQRT2 = 1.0 / math.sqrt(2.0)
_NEG_INF = -1e30
_HEADS = 12
_PATCH = 16
_EPS = 1e-5


def _ru(x, m):
    return ((x + m - 1) // m) * m


def _vmem_limit(bytes_needed):
    return int(min(64 * 2**20, max(32 * 2**20, 2 * bytes_needed)))


def _ln_rows(xv, g, b):
    mean = jnp.mean(xv, axis=-1, keepdims=True)
    xc = xv - mean
    var = jnp.mean(xc * xc, axis=-1, keepdims=True)
    return xc * lax.rsqrt(var + _EPS) * g + b


def _embed_kernel(x_ref, w_ref, b_ref, pos_ref, cls_ref, o_ref,
                  *, n_tok, n_pad, gh, p):
    xv = x_ref[0]
    c_in = xv.shape[0]
    p6 = xv.reshape(c_in, gh, p, gh, p)
    patches = p6.transpose(1, 3, 0, 2, 4).reshape(gh * gh, c_in * p * p)
    t = jnp.dot(patches.astype(jnp.bfloat16), w_ref[...],
                preferred_element_type=jnp.float32)
    t = jnp.pad(t, ((1, n_pad - n_tok), (0, 0)))
    y = t + b_ref[...] + pos_ref[...]
    rows = lax.broadcasted_iota(jnp.int32, (y.shape[0], 1), 0)
    y = jnp.where(rows == 0, cls_ref[...], y)
    y = jnp.where(rows >= n_tok, 0.0, y)
    o_ref[0] = y


def _mha(qkv, mask, *, nb, n_pad, heads, hd, scale):
    dim = heads * hd
    rows_out = []
    for bi in range(nb):
        r0 = bi * n_pad
        heads_out = []
        for h in range(heads):
            q = qkv[r0:r0 + n_pad, h * hd:(h + 1) * hd].astype(jnp.bfloat16)
            k = qkv[r0:r0 + n_pad,
                    dim + h * hd:dim + (h + 1) * hd].astype(jnp.bfloat16)
            v = qkv[r0:r0 + n_pad,
                    2 * dim + h * hd:2 * dim + (h + 1) * hd].astype(jnp.bfloat16)
            s = lax.dot_general(q, k, (((1,), (1,)), ((), ())),
                                preferred_element_type=jnp.float32) * scale
            s = jnp.where(mask, _NEG_INF, s)
            s = s - jnp.max(s, axis=-1, keepdims=True)
            p = jnp.exp(s)
            p = p / jnp.sum(p, axis=-1, keepdims=True)
            heads_out.append(jnp.dot(p.astype(jnp.bfloat16), v,
                                     preferred_element_type=jnp.float32))
        rows_out.append(jnp.concatenate(heads_out, axis=1))
    return jnp.concatenate(rows_out, axis=0)


def _attn_kernel(x_ref, g_ref, b_ref, wq_ref, bq_ref, o_ref,
                 *, nb, n_pad, n_tok, heads, hd, scale):
    xv = x_ref[...].reshape(nb * n_pad, x_ref.shape[-1])
    ln = _ln_rows(xv, g_ref[...], b_ref[...]).astype(jnp.bfloat16)
    qkv = jnp.dot(ln, wq_ref[...], preferred_element_type=jnp.float32)
    qkv = qkv + bq_ref[...]
    mask = lax.broadcasted_iota(jnp.int32, (n_pad, n_pad), 1) >= n_tok
    o = _mha(qkv, mask, nb=nb, n_pad=n_pad, heads=heads, hd=hd, scale=scale)
    o_ref[...] = o.astype(jnp.bfloat16).reshape(o_ref.shape)


def _block_kernel(o_ref, x_ref, pw_ref, pb_ref, g_ref, b_ref,
                  w1_ref, b1_ref, w2_ref, b2_ref, out_ref,
                  xmid_ref, ln_ref, acc_ref):
    k = pl.program_id(1)

    @pl.when(k == 0)
    def _():
        rows = o_ref.shape[0] * o_ref.shape[1]
        ov = o_ref[...].reshape(rows, o_ref.shape[-1])
        xv = x_ref[...].reshape(rows, x_ref.shape[-1])
        t = jnp.dot(ov, pw_ref[...],
                    preferred_element_type=jnp.float32) + pb_ref[...]
        xmid = xv + t
        xmid_ref[...] = xmid
        ln_ref[...] = _ln_rows(xmid, g_ref[...], b_ref[...]).astype(jnp.bfloat16)
        acc_ref[...] = jnp.zeros_like(acc_ref)

    hh = jnp.dot(ln_ref[...], w1_ref[...],
                 preferred_element_type=jnp.float32) + b1_ref[0]
    gl = 0.5 * hh * (1.0 + lax.erf(hh * _INV_SQRT2))
    acc_ref[...] += jnp.dot(gl.astype(jnp.bfloat16), w2_ref[...],
                            preferred_element_type=jnp.float32)

    @pl.when(k == pl.num_programs(1) - 1)
    def _():
        out_ref[...] = (xmid_ref[...] + acc_ref[...]
                        + b2_ref[...]).reshape(out_ref.shape)


def _final_kernel(x_ref, g_ref, b_ref, o_ref):
    o_ref[...] = _ln_rows(x_ref[...], g_ref[...], b_ref[...])


def _row2d(a):
    return a.reshape(1, a.shape[-1]).astype(jnp.float32)


def kernel(patch_embed_w, patch_embed_b, cls_token, pos_embed, norm_g, norm_b, block0_ln1_g, block0_ln1_b, block0_qkv_w, block0_qkv_b, block0_proj_w, block0_proj_b, block0_ln2_g, block0_ln2_b, block0_fc1_w, block0_fc1_b, block0_fc2_w, block0_fc2_b, block1_ln1_g, block1_ln1_b, block1_qkv_w, block1_qkv_b, block1_proj_w, block1_proj_b, block1_ln2_g, block1_ln2_b, block1_fc1_w, block1_fc1_b, block1_fc2_w, block1_fc2_b, block2_ln1_g, block2_ln1_b, block2_qkv_w, block2_qkv_b, block2_proj_w, block2_proj_b, block2_ln2_g, block2_ln2_b, block2_fc1_w, block2_fc1_b, block2_fc2_w, block2_fc2_b, block3_ln1_g, block3_ln1_b, block3_qkv_w, block3_qkv_b, block3_proj_w, block3_proj_b, block3_ln2_g, block3_ln2_b, block3_fc1_w, block3_fc1_b, block3_fc2_w, block3_fc2_b, block4_ln1_g, block4_ln1_b, block4_qkv_w, block4_qkv_b, block4_proj_w, block4_proj_b, block4_ln2_g, block4_ln2_b, block4_fc1_w, block4_fc1_b, block4_fc2_w, block4_fc2_b, block5_ln1_g, block5_ln1_b, block5_qkv_w, block5_qkv_b, block5_proj_w, block5_proj_b, block5_ln2_g, block5_ln2_b, block5_fc1_w, block5_fc1_b, block5_fc2_w, block5_fc2_b, x):
    blocks = [
        (block0_ln1_g, block0_ln1_b, block0_qkv_w, block0_qkv_b, block0_proj_w,
         block0_proj_b, block0_ln2_g, block0_ln2_b, block0_fc1_w, block0_fc1_b,
         block0_fc2_w, block0_fc2_b),
        (block1_ln1_g, block1_ln1_b, block1_qkv_w, block1_qkv_b, block1_proj_w,
         block1_proj_b, block1_ln2_g, block1_ln2_b, block1_fc1_w, block1_fc1_b,
         block1_fc2_w, block1_fc2_b),
        (block2_ln1_g, block2_ln1_b, block2_qkv_w, block2_qkv_b, block2_proj_w,
         block2_proj_b, block2_ln2_g, block2_ln2_b, block2_fc1_w, block2_fc1_b,
         block2_fc2_w, block2_fc2_b),
        (block3_ln1_g, block3_ln1_b, block3_qkv_w, block3_qkv_b, block3_proj_w,
         block3_proj_b, block3_ln2_g, block3_ln2_b, block3_fc1_w, block3_fc1_b,
         block3_fc2_w, block3_fc2_b),
        (block4_ln1_g, block4_ln1_b, block4_qkv_w, block4_qkv_b, block4_proj_w,
         block4_proj_b, block4_ln2_g, block4_ln2_b, block4_fc1_w, block4_fc1_b,
         block4_fc2_w, block4_fc2_b),
        (block5_ln1_g, block5_ln1_b, block5_qkv_w, block5_qkv_b, block5_proj_w,
         block5_proj_b, block5_ln2_g, block5_ln2_b, block5_fc1_w, block5_fc1_b,
         block5_fc2_w, block5_fc2_b),
    ]

    B, C, IMG, _ = x.shape
    p = _PATCH
    gh = IMG // p
    n_patch = gh * gh
    n_tok = n_patch + 1
    n_pad = _ru(n_tok, 8)
    D = patch_embed_w.shape[1]
    K = C * p * p
    H = _HEADS
    hd = D // H
    hidden = blocks[0][8].shape[1]
    scale = hd ** -0.5

    pos = pos_embed[0].astype(jnp.float32)
    pos_pad = jnp.pad(pos, ((0, n_pad - n_tok), (0, 0)))
    cls0 = (cls_token[0, 0] + pos[0]).reshape(1, D).astype(jnp.float32)

    xs = pl.pallas_call(
        partial(_embed_kernel, n_tok=n_tok, n_pad=n_pad, gh=gh, p=p),
        out_shape=jax.ShapeDtypeStruct((B, n_pad, D), jnp.float32),
        grid_spec=pltpu.PrefetchScalarGridSpec(
            num_scalar_prefetch=0,
            grid=(B,),
            in_specs=[
                pl.BlockSpec((1, C, IMG, IMG), lambda i: (i, 0, 0, 0)),
                pl.BlockSpec((K, D), lambda i: (0, 0)),
                pl.BlockSpec((1, D), lambda i: (0, 0)),
                pl.BlockSpec((n_pad, D), lambda i: (0, 0)),
                pl.BlockSpec((1, D), lambda i: (0, 0)),
            ],
            out_specs=pl.BlockSpec((1, n_pad, D), lambda i: (i, 0, 0)),
        ),
        compiler_params=pltpu.CompilerParams(
            dimension_semantics=("parallel",),
            vmem_limit_bytes=_vmem_limit(4 * (K * D + 3 * n_pad * D + C * IMG * IMG))),
    )(x, patch_embed_w.astype(jnp.bfloat16), _row2d(patch_embed_b), pos_pad, cls0)

    NB_A = min(2, B)
    NB_B = min(4, B)
    KS = 4
    ts = hidden // KS
    aspec = pl.BlockSpec((NB_A, n_pad, D), lambda i: (i, 0, 0))
    arow = pl.BlockSpec((1, D), lambda i: (0, 0))
    bspec = pl.BlockSpec((NB_B, n_pad, D), lambda i, k: (i, 0, 0))
    brow = pl.BlockSpec((1, D), lambda i, k: (0, 0))
    for (ln1_g, ln1_b, qkv_w, qkv_b, proj_w, proj_b,
         ln2_g, ln2_b, fc1_w, fc1_b, fc2_w, fc2_b) in blocks:
        o_t = pl.pallas_call(
            partial(_attn_kernel, nb=NB_A, n_pad=n_pad, n_tok=n_tok,
                    heads=H, hd=hd, scale=scale),
            out_shape=jax.ShapeDtypeStruct((B, n_pad, D), jnp.bfloat16),
            grid_spec=pltpu.PrefetchScalarGridSpec(
                num_scalar_prefetch=0,
                grid=(B // NB_A,),
                in_specs=[
                    aspec,
                    arow,
                    arow,
                    pl.BlockSpec((D, 3 * D), lambda i: (0, 0)),
                    pl.BlockSpec((1, 3 * D), lambda i: (0, 0)),
                ],
                out_specs=aspec,
            ),
            compiler_params=pltpu.CompilerParams(
                dimension_semantics=("parallel",),
                vmem_limit_bytes=_vmem_limit(
                    2 * D * 3 * D + 4 * NB_A * (3 * n_pad * D + n_pad * 3 * D
                                                + 2 * n_pad * n_pad))),
        )(xs, _row2d(ln1_g), _row2d(ln1_b),
          qkv_w.astype(jnp.bfloat16), _row2d(qkv_b))
        xs = pl.pallas_call(
            _block_kernel,
            out_shape=jax.ShapeDtypeStruct((B, n_pad, D), jnp.float32),
            grid_spec=pltpu.PrefetchScalarGridSpec(
                num_scalar_prefetch=0,
                grid=(B // NB_B, KS),
                in_specs=[
                    bspec,
                    bspec,
                    pl.BlockSpec((D, D), lambda i, k: (0, 0)),
                    brow,
                    brow,
                    brow,
                    pl.BlockSpec((D, ts), lambda i, k: (0, k)),
                    pl.BlockSpec((1, 1, ts), lambda i, k: (k, 0, 0)),
                    pl.BlockSpec((ts, D), lambda i, k: (k, 0)),
                    brow,
                ],
                out_specs=pl.BlockSpec((NB_B, n_pad, D), lambda i, k: (i, 0, 0)),
                scratch_shapes=[
                    pltpu.VMEM((NB_B * n_pad, D), jnp.float32),
                    pltpu.VMEM((NB_B * n_pad, D), jnp.bfloat16),
                    pltpu.VMEM((NB_B * n_pad, D), jnp.float32),
                ],
            ),
            compiler_params=pltpu.CompilerParams(
                dimension_semantics=("parallel", "arbitrary"),
                vmem_limit_bytes=_vmem_limit(
                    2 * (D * D + 2 * D * ts) + 10 * NB_B * n_pad * D
                    + 4 * NB_B * (2 * n_pad * D + n_pad * ts))),
        )(o_t, xs, proj_w.astype(jnp.bfloat16), _row2d(proj_b),
          _row2d(ln2_g), _row2d(ln2_b), fc1_w.astype(jnp.bfloat16),
          fc1_b.reshape(KS, 1, ts).astype(jnp.float32),
          fc2_w.astype(jnp.bfloat16), _row2d(fc2_b))

    cls_rows = xs[:, 0, :]
    out = pl.pallas_call(
        _final_kernel,
        out_shape=jax.ShapeDtypeStruct((B, D), jnp.float32),
        grid_spec=pltpu.PrefetchScalarGridSpec(
            num_scalar_prefetch=0,
            grid=(1,),
            in_specs=[
                pl.BlockSpec((B, D), lambda i: (0, 0)),
                pl.BlockSpec((1, D), lambda i: (0, 0)),
                pl.BlockSpec((1, D), lambda i: (0, 0)),
            ],
            out_specs=pl.BlockSpec((B, D), lambda i: (0, 0)),
        ),
        compiler_params=pltpu.CompilerParams(
            dimension_semantics=("arbitrary",)),
    )(cls_rows, _row2d(norm_g), _row2d(norm_b))
    return out


# speedup vs baseline: 1.3432x; 1.0011x over previous
import math
from functools import partial

import jax
import jax.numpy as jnp
from jax import lax
from jax.experimental import pallas as pl
from jax.experimental.pallas import tpu as pltpu

_INV_SQRT2 = 1.0 / math.sqrt(2.0)
_NEG_INF = -1e30
_HEADS = 12
_PATCH = 16
_EPS = 1e-5


def _ru(x, m):
    return ((x + m - 1) // m) * m


def _vmem_limit(bytes_needed):
    return int(min(64 * 2**20, max(32 * 2**20, 2 * bytes_needed)))


def _ln_rows(xv, g, b):
    mean = jnp.mean(xv, axis=-1, keepdims=True)
    xc = xv - mean
    var = jnp.mean(xc * xc, axis=-1, keepdims=True)
    return xc * lax.rsqrt(var + _EPS) * g + b


def _embed_kernel(x_ref, w_ref, b_ref, pos_ref, cls_ref, o_ref,
                  *, nb, n_tok, n_pad, gh, p):
    c_in = x_ref.shape[1]
    pats = []
    for bi in range(nb):
        p6 = x_ref[bi].reshape(c_in, gh, p, gh, p)
        pats.append(p6.transpose(1, 3, 0, 2, 4).reshape(gh * gh, c_in * p * p))
    t = jnp.dot(jnp.concatenate(pats, axis=0).astype(jnp.bfloat16), w_ref[...],
                preferred_element_type=jnp.float32)
    rows = lax.broadcasted_iota(jnp.int32, (n_pad, 1), 0)
    for bi in range(nb):
        tb = jnp.pad(t[bi * gh * gh:(bi + 1) * gh * gh],
                     ((1, n_pad - n_tok), (0, 0)))
        y = tb + b_ref[...] + pos_ref[...]
        y = jnp.where(rows == 0, cls_ref[...], y)
        y = jnp.where(rows >= n_tok, 0.0, y)
        o_ref[bi] = y


def _mha(qkv, mask, *, nb, n_pad, heads, hd, scale):
    dim = heads * hd
    rows_out = []
    for bi in range(nb):
        r0 = bi * n_pad
        heads_out = []
        for h in range(heads):
            q = qkv[r0:r0 + n_pad, h * hd:(h + 1) * hd].astype(jnp.bfloat16)
            k = qkv[r0:r0 + n_pad,
                    dim + h * hd:dim + (h + 1) * hd].astype(jnp.bfloat16)
            v = qkv[r0:r0 + n_pad,
                    2 * dim + h * hd:2 * dim + (h + 1) * hd].astype(jnp.bfloat16)
            s = lax.dot_general(q, k, (((1,), (1,)), ((), ())),
                                preferred_element_type=jnp.float32) * scale
            s = jnp.where(mask, _NEG_INF, s)
            s = s - jnp.max(s, axis=-1, keepdims=True)
            p = jnp.exp(s)
            p = p / jnp.sum(p, axis=-1, keepdims=True)
            heads_out.append(jnp.dot(p.astype(jnp.bfloat16), v,
                                     preferred_element_type=jnp.float32))
        rows_out.append(jnp.concatenate(heads_out, axis=1))
    return jnp.concatenate(rows_out, axis=0)


def _attn_kernel(x_ref, g_ref, b_ref, wq_ref, bq_ref, o_ref,
                 *, nb, n_pad, n_tok, heads, hd, scale):
    xv = x_ref[...].reshape(nb * n_pad, x_ref.shape[-1])
    ln = _ln_rows(xv, g_ref[...], b_ref[...]).astype(jnp.bfloat16)
    qkv = jnp.dot(ln, wq_ref[...], preferred_element_type=jnp.float32)
    qkv = qkv + bq_ref[...]
    mask = lax.broadcasted_iota(jnp.int32, (n_pad, n_pad), 1) >= n_tok
    o = _mha(qkv, mask, nb=nb, n_pad=n_pad, heads=heads, hd=hd, scale=scale)
    o_ref[...] = o.astype(jnp.bfloat16).reshape(o_ref.shape)


def _block_kernel(o_ref, x_ref, pw_ref, pb_ref, g_ref, b_ref,
                  w1_ref, b1_ref, w2_ref, b2_ref, out_ref,
                  xmid_ref, ln_ref, acc_ref):
    k = pl.program_id(1)

    @pl.when(k == 0)
    def _():
        rows = o_ref.shape[0] * o_ref.shape[1]
        ov = o_ref[...].reshape(rows, o_ref.shape[-1])
        xv = x_ref[...].reshape(rows, x_ref.shape[-1])
        t = jnp.dot(ov, pw_ref[...],
                    preferred_element_type=jnp.float32) + pb_ref[...]
        xmid = xv + t
        xmid_ref[...] = xmid
        ln_ref[...] = _ln_rows(xmid, g_ref[...], b_ref[...]).astype(jnp.bfloat16)
        acc_ref[...] = jnp.zeros_like(acc_ref)

    hh = jnp.dot(ln_ref[...], w1_ref[...],
                 preferred_element_type=jnp.float32) + b1_ref[0]
    gl = 0.5 * hh * (1.0 + lax.erf(hh * _INV_SQRT2))
    acc_ref[...] += jnp.dot(gl.astype(jnp.bfloat16), w2_ref[...],
                            preferred_element_type=jnp.float32)

    @pl.when(k == pl.num_programs(1) - 1)
    def _():
        out_ref[...] = (xmid_ref[...] + acc_ref[...]
                        + b2_ref[...]).reshape(out_ref.shape)


def _final_kernel(x_ref, g_ref, b_ref, o_ref):
    o_ref[...] = _ln_rows(x_ref[...], g_ref[...], b_ref[...])


def _row2d(a):
    return a.reshape(1, a.shape[-1]).astype(jnp.float32)


def kernel(patch_embed_w, patch_embed_b, cls_token, pos_embed, norm_g, norm_b, block0_ln1_g, block0_ln1_b, block0_qkv_w, block0_qkv_b, block0_proj_w, block0_proj_b, block0_ln2_g, block0_ln2_b, block0_fc1_w, block0_fc1_b, block0_fc2_w, block0_fc2_b, block1_ln1_g, block1_ln1_b, block1_qkv_w, block1_qkv_b, block1_proj_w, block1_proj_b, block1_ln2_g, block1_ln2_b, block1_fc1_w, block1_fc1_b, block1_fc2_w, block1_fc2_b, block2_ln1_g, block2_ln1_b, block2_qkv_w, block2_qkv_b, block2_proj_w, block2_proj_b, block2_ln2_g, block2_ln2_b, block2_fc1_w, block2_fc1_b, block2_fc2_w, block2_fc2_b, block3_ln1_g, block3_ln1_b, block3_qkv_w, block3_qkv_b, block3_proj_w, block3_proj_b, block3_ln2_g, block3_ln2_b, block3_fc1_w, block3_fc1_b, block3_fc2_w, block3_fc2_b, block4_ln1_g, block4_ln1_b, block4_qkv_w, block4_qkv_b, block4_proj_w, block4_proj_b, block4_ln2_g, block4_ln2_b, block4_fc1_w, block4_fc1_b, block4_fc2_w, block4_fc2_b, block5_ln1_g, block5_ln1_b, block5_qkv_w, block5_qkv_b, block5_proj_w, block5_proj_b, block5_ln2_g, block5_ln2_b, block5_fc1_w, block5_fc1_b, block5_fc2_w, block5_fc2_b, x):
    blocks = [
        (block0_ln1_g, block0_ln1_b, block0_qkv_w, block0_qkv_b, block0_proj_w,
         block0_proj_b, block0_ln2_g, block0_ln2_b, block0_fc1_w, block0_fc1_b,
         block0_fc2_w, block0_fc2_b),
        (block1_ln1_g, block1_ln1_b, block1_qkv_w, block1_qkv_b, block1_proj_w,
         block1_proj_b, block1_ln2_g, block1_ln2_b, block1_fc1_w, block1_fc1_b,
         block1_fc2_w, block1_fc2_b),
        (block2_ln1_g, block2_ln1_b, block2_qkv_w, block2_qkv_b, block2_proj_w,
         block2_proj_b, block2_ln2_g, block2_ln2_b, block2_fc1_w, block2_fc1_b,
         block2_fc2_w, block2_fc2_b),
        (block3_ln1_g, block3_ln1_b, block3_qkv_w, block3_qkv_b, block3_proj_w,
         block3_proj_b, block3_ln2_g, block3_ln2_b, block3_fc1_w, block3_fc1_b,
         block3_fc2_w, block3_fc2_b),
        (block4_ln1_g, block4_ln1_b, block4_qkv_w, block4_qkv_b, block4_proj_w,
         block4_proj_b, block4_ln2_g, block4_ln2_b, block4_fc1_w, block4_fc1_b,
         block4_fc2_w, block4_fc2_b),
        (block5_ln1_g, block5_ln1_b, block5_qkv_w, block5_qkv_b, block5_proj_w,
         block5_proj_b, block5_ln2_g, block5_ln2_b, block5_fc1_w, block5_fc1_b,
         block5_fc2_w, block5_fc2_b),
    ]

    B, C, IMG, _ = x.shape
    p = _PATCH
    gh = IMG // p
    n_patch = gh * gh
    n_tok = n_patch + 1
    n_pad = _ru(n_tok, 8)
    D = patch_embed_w.shape[1]
    K = C * p * p
    H = _HEADS
    hd = D // H
    hidden = blocks[0][8].shape[1]
    scale = hd ** -0.5

    pos = pos_embed[0].astype(jnp.float32)
    pos_pad = jnp.pad(pos, ((0, n_pad - n_tok), (0, 0)))
    cls0 = (cls_token[0, 0] + pos[0]).reshape(1, D).astype(jnp.float32)

    NB_E = min(2, B)
    xs = pl.pallas_call(
        partial(_embed_kernel, nb=NB_E, n_tok=n_tok, n_pad=n_pad, gh=gh, p=p),
        out_shape=jax.ShapeDtypeStruct((B, n_pad, D), jnp.float32),
        grid_spec=pltpu.PrefetchScalarGridSpec(
            num_scalar_prefetch=0,
            grid=(B // NB_E,),
            in_specs=[
                pl.BlockSpec((NB_E, C, IMG, IMG), lambda i: (i, 0, 0, 0)),
                pl.BlockSpec((K, D), lambda i: (0, 0)),
                pl.BlockSpec((1, D), lambda i: (0, 0)),
                pl.BlockSpec((n_pad, D), lambda i: (0, 0)),
                pl.BlockSpec((1, D), lambda i: (0, 0)),
            ],
            out_specs=pl.BlockSpec((NB_E, n_pad, D), lambda i: (i, 0, 0)),
        ),
        compiler_params=pltpu.CompilerParams(
            dimension_semantics=("parallel",),
            vmem_limit_bytes=_vmem_limit(4 * (K * D + 3 * n_pad * D + C * IMG * IMG))),
    )(x, patch_embed_w.astype(jnp.bfloat16), _row2d(patch_embed_b), pos_pad, cls0)

    NB_A = min(2, B)
    NB_B = min(4, B)
    KS = 4
    ts = hidden // KS
    aspec = pl.BlockSpec((NB_A, n_pad, D), lambda i: (i, 0, 0))
    arow = pl.BlockSpec((1, D), lambda i: (0, 0))
    bspec = pl.BlockSpec((NB_B, n_pad, D), lambda i, k: (i, 0, 0))
    brow = pl.BlockSpec((1, D), lambda i, k: (0, 0))
    for (ln1_g, ln1_b, qkv_w, qkv_b, proj_w, proj_b,
         ln2_g, ln2_b, fc1_w, fc1_b, fc2_w, fc2_b) in blocks:
        o_t = pl.pallas_call(
            partial(_attn_kernel, nb=NB_A, n_pad=n_pad, n_tok=n_tok,
                    heads=H, hd=hd, scale=scale),
            out_shape=jax.ShapeDtypeStruct((B, n_pad, D), jnp.bfloat16),
            grid_spec=pltpu.PrefetchScalarGridSpec(
                num_scalar_prefetch=0,
                grid=(B // NB_A,),
                in_specs=[
                    aspec,
                    arow,
                    arow,
                    pl.BlockSpec((D, 3 * D), lambda i: (0, 0)),
                    pl.BlockSpec((1, 3 * D), lambda i: (0, 0)),
                ],
                out_specs=aspec,
            ),
            compiler_params=pltpu.CompilerParams(
                dimension_semantics=("parallel",),
                vmem_limit_bytes=_vmem_limit(
                    2 * D * 3 * D + 4 * NB_A * (3 * n_pad * D + n_pad * 3 * D
                                                + 2 * n_pad * n_pad))),
        )(xs, _row2d(ln1_g), _row2d(ln1_b),
          qkv_w.astype(jnp.bfloat16), _row2d(qkv_b))
        xs = pl.pallas_call(
            _block_kernel,
            out_shape=jax.ShapeDtypeStruct((B, n_pad, D), jnp.float32),
            grid_spec=pltpu.PrefetchScalarGridSpec(
                num_scalar_prefetch=0,
                grid=(B // NB_B, KS),
                in_specs=[
                    bspec,
                    bspec,
                    pl.BlockSpec((D, D), lambda i, k: (0, 0)),
                    brow,
                    brow,
                    brow,
                    pl.BlockSpec((D, ts), lambda i, k: (0, k)),
                    pl.BlockSpec((1, 1, ts), lambda i, k: (k, 0, 0)),
                    pl.BlockSpec((ts, D), lambda i, k: (k, 0)),
                    brow,
                ],
                out_specs=pl.BlockSpec((NB_B, n_pad, D), lambda i, k: (i, 0, 0)),
                scratch_shapes=[
                    pltpu.VMEM((NB_B * n_pad, D), jnp.float32),
                    pltpu.VMEM((NB_B * n_pad, D), jnp.bfloat16),
                    pltpu.VMEM((NB_B * n_pad, D), jnp.float32),
                ],
            ),
            compiler_params=pltpu.CompilerParams(
                dimension_semantics=("parallel", "arbitrary"),
                vmem_limit_bytes=_vmem_limit(
                    2 * (D * D + 2 * D * ts) + 10 * NB_B * n_pad * D
                    + 4 * NB_B * (2 * n_pad * D + n_pad * ts))),
        )(o_t, xs, proj_w.astype(jnp.bfloat16), _row2d(proj_b),
          _row2d(ln2_g), _row2d(ln2_b), fc1_w.astype(jnp.bfloat16),
          fc1_b.reshape(KS, 1, ts).astype(jnp.float32),
          fc2_w.astype(jnp.bfloat16), _row2d(fc2_b))

    cls_rows = xs[:, 0, :]
    out = pl.pallas_call(
        _final_kernel,
        out_shape=jax.ShapeDtypeStruct((B, D), jnp.float32),
        grid_spec=pltpu.PrefetchScalarGridSpec(
            num_scalar_prefetch=0,
            grid=(1,),
            in_specs=[
                pl.BlockSpec((B, D), lambda i: (0, 0)),
                pl.BlockSpec((1, D), lambda i: (0, 0)),
                pl.BlockSpec((1, D), lambda i: (0, 0)),
            ],
            out_specs=pl.BlockSpec((B, D), lambda i: (0, 0)),
        ),
        compiler_params=pltpu.CompilerParams(
            dimension_semantics=("arbitrary",)),
    )(cls_rows, _row2d(norm_g), _row2d(norm_b))
    return out
